# Initial kernel scaffold; baseline (speedup 1.0000x reference)
import jax
import jax.numpy as jnp
from jax.experimental import pallas as pl


def kernel(Z, Rij, idx_i, idx_j, emb, Win2f, Wf1, bf1, Wf2, bf2, Wo1, bo1, Wo2, bo2):
    raise NotImplementedError("write your pallas kernel here")


# speedup vs baseline: 1.9314x; 1.9314x over previous
import functools

import jax
import jax.numpy as jnp
from jax import lax
from jax.experimental import pallas as pl
from jax.experimental.pallas import tpu as pltpu
from jax.experimental.pallas import tpu_sc as plsc

N = 10000
E = 320000
D = 128
NRBF = 20
CUTOFF = 5.0
NI = 3
MAX_Z = 100

MZP = 104
NRBFP = 24
LOG2 = 0.6931471805599453

NCORES = 2
NSUB = 16
NW = NCORES * NSUB
EPW = E // NW
CHUNK = 80
NCHUNK = EPW // CHUNK
NPAD = 10240
STRIPE = NPAD // NSUB
BN = 1000
BE = 1000


def _ssp(x):
    return jnp.maximum(x, 0.0) + jnp.log(1.0 + jnp.exp(-jnp.abs(x))) - LOG2


def _embed_body(z_ref, emb_ref, out_ref):
    z = z_ref[...]
    col = lax.broadcasted_iota(jnp.int32, (BN, MZP), 1)
    oh = (z == col).astype(jnp.float32)
    out_ref[...] = jnp.dot(oh, emb_ref[...], preferred_element_type=jnp.float32)


def _embed(z2, embp):
    return pl.pallas_call(
        _embed_body,
        grid=(N // BN,),
        in_specs=[
            pl.BlockSpec((BN, 1), lambda i: (i, 0)),
            pl.BlockSpec((MZP, D), lambda i: (0, 0)),
        ],
        out_specs=pl.BlockSpec((BN, D), lambda i: (i, 0)),
        out_shape=jax.ShapeDtypeStruct((N, D), jnp.float32),
    )(z2, embp)


def _mm_body(x_ref, w_ref, out_ref):
    out_ref[...] = jnp.dot(x_ref[...], w_ref[...], preferred_element_type=jnp.float32)


def _in2f(x, w):
    return pl.pallas_call(
        _mm_body,
        grid=(N // BN,),
        in_specs=[
            pl.BlockSpec((BN, D), lambda i: (i, 0)),
            pl.BlockSpec((D, D), lambda i: (0, 0)),
        ],
        out_specs=pl.BlockSpec((BN, D), lambda i: (i, 0)),
        out_shape=jax.ShapeDtypeStruct((N, D), jnp.float32),
    )(x, w)


def _filter_body(r_ref, wf1_ref, bf1_ref, wf2_ref, bf2_ref, out_ref):
    r = r_ref[...]
    d2 = jnp.sum(r * r, axis=1, keepdims=True)
    d = jnp.sqrt(d2)
    delta = CUTOFF / (NRBF - 1)
    offs = delta * lax.broadcasted_iota(jnp.int32, (1, NRBFP), 1).astype(jnp.float32)
    coeff = -0.5 / (delta * delta)
    f = jnp.exp(coeff * (d - offs) ** 2)
    h = _ssp(jnp.dot(f, wf1_ref[...], preferred_element_type=jnp.float32)
             + bf1_ref[...])
    w = jnp.dot(h, wf2_ref[...], preferred_element_type=jnp.float32) + bf2_ref[...]
    rcut = 0.5 * (jnp.cos(d * (jnp.pi / CUTOFF)) + 1.0)
    rcut = rcut * (d < CUTOFF).astype(jnp.float32)
    out_ref[...] = w * rcut


def _filter(rij, wf1p, bf1, wf2, bf2):
    return pl.pallas_call(
        _filter_body,
        grid=(E // BE,),
        in_specs=[
            pl.BlockSpec((BE, 3), lambda i: (i, 0)),
            pl.BlockSpec((NRBFP, D), lambda i: (0, 0)),
            pl.BlockSpec((1, D), lambda i: (0, 0)),
            pl.BlockSpec((D, D), lambda i: (0, 0)),
            pl.BlockSpec((1, D), lambda i: (0, 0)),
        ],
        out_specs=pl.BlockSpec((BE, D), lambda i: (i, 0)),
        out_shape=jax.ShapeDtypeStruct((E, D), jnp.float32),
    )(rij, wf1p, bf1, wf2, bf2)


def _out_body(agg_ref, x_ref, w1_ref, b1_ref, w2_ref, b2_ref, out_ref):
    agg = agg_ref[0] + agg_ref[1]
    h = _ssp(jnp.dot(agg, w1_ref[...], preferred_element_type=jnp.float32)
             + b1_ref[...])
    v = jnp.dot(h, w2_ref[...], preferred_element_type=jnp.float32) + b2_ref[...]
    out_ref[...] = x_ref[...] + v


def _out(agg_p, x, w1, b1, w2, b2):
    return pl.pallas_call(
        _out_body,
        grid=(N // BN,),
        in_specs=[
            pl.BlockSpec((2, BN, D), lambda i: (0, i, 0)),
            pl.BlockSpec((BN, D), lambda i: (i, 0)),
            pl.BlockSpec((D, D), lambda i: (0, 0)),
            pl.BlockSpec((1, D), lambda i: (0, 0)),
            pl.BlockSpec((D, D), lambda i: (0, 0)),
            pl.BlockSpec((1, D), lambda i: (0, 0)),
        ],
        out_specs=pl.BlockSpec((BN, D), lambda i: (i, 0)),
        out_shape=jax.ShapeDtypeStruct((N, D), jnp.float32),
    )(agg_p, x, w1, b1, w2, b2)


def _sc_edge_body(xf_hbm, wij_hbm, idxi_hbm, idxj_hbm, out_hbm,
                  idxi_v, idxj_v, rows_v, wij_v, agg_sh, sem):
    cid = lax.axis_index("c")
    sid = lax.axis_index("s")
    wid = cid * NSUB + sid

    zeros16 = jnp.zeros((16,), jnp.float32)

    def _zero_row(e, _):
        for k in range(D // 16):
            wij_v[e, pl.ds(k * 16, 16)] = zeros16
        return ()

    lax.fori_loop(0, CHUNK, _zero_row, ())
    for t in range(STRIPE // CHUNK):
        pltpu.sync_copy(wij_v, agg_sh.at[pl.ds(sid * STRIPE + t * CHUNK, CHUNK)])
    plsc.subcore_barrier()

    def _chunk(t, _):
        base = wid * EPW + t * CHUNK
        pltpu.sync_copy(idxj_hbm.at[pl.ds(base, CHUNK)], idxj_v)
        pltpu.sync_copy(idxi_hbm.at[pl.ds(base, CHUNK)], idxi_v)
        pltpu.async_copy(xf_hbm.at[idxj_v], rows_v, sem).wait()
        pltpu.sync_copy(wij_hbm.at[pl.ds(base, CHUNK)], wij_v)

        def _mul(e, _):
            for k in range(D // 16):
                s = pl.ds(k * 16, 16)
                rows_v[e, s] = rows_v[e, s] * wij_v[e, s]
            return ()

        lax.fori_loop(0, CHUNK, _mul, ())
        pltpu.sync_copy(rows_v, agg_sh.at[idxi_v], add=True)
        return ()

    lax.fori_loop(0, NCHUNK, _chunk, ())
    plsc.subcore_barrier()
    pltpu.sync_copy(agg_sh.at[pl.ds(sid * STRIPE, STRIPE)],
                    out_hbm.at[cid, pl.ds(sid * STRIPE, STRIPE)])


_sc_edge_built = None


def _sc_edge(xf, wij, idx_i, idx_j):
    global _sc_edge_built
    if _sc_edge_built is None:
        mesh = plsc.VectorSubcoreMesh(core_axis_name="c", subcore_axis_name="s")
        _sc_edge_built = pl.kernel(
            _sc_edge_body,
            mesh=mesh,
            out_type=jax.ShapeDtypeStruct((NCORES, NPAD, D), jnp.float32),
            scratch_types=[
                pltpu.VMEM((CHUNK,), jnp.int32),
                pltpu.VMEM((CHUNK,), jnp.int32),
                pltpu.VMEM((CHUNK, D), jnp.float32),
                pltpu.VMEM((CHUNK, D), jnp.float32),
                pltpu.VMEM_SHARED((NPAD, D), jnp.float32),
                pltpu.SemaphoreType.DMA,
            ],
        )
    return _sc_edge_built(xf, wij, idx_i, idx_j)


def kernel(Z, Rij, idx_i, idx_j, emb, Win2f, Wf1, bf1, Wf2, bf2, Wo1, bo1, Wo2, bo2):
    embp = jnp.zeros((MZP, D), jnp.float32).at[:MAX_Z].set(emb)
    x = _embed(Z.reshape(N, 1).astype(jnp.int32), embp)
    idx_i = idx_i.astype(jnp.int32)
    idx_j = idx_j.astype(jnp.int32)
    for i in range(NI):
        xf = _in2f(x, Win2f[i])
        wf1p = jnp.zeros((NRBFP, D), jnp.float32).at[:NRBF].set(Wf1[i])
        wij = _filter(Rij, wf1p, bf1[i][None], Wf2[i], bf2[i][None])
        agg_p = _sc_edge(xf, wij, idx_i, idx_j)
        x = _out(agg_p, x, Wo1[i], bo1[i][None], Wo2[i], bo2[i][None])
    return x
